# baseline (speedup 1.0000x reference)
import jax
import jax.numpy as jnp
from jax import lax
from jax.experimental import pallas as pl
from jax.experimental.pallas import tpu as pltpu
from jax.experimental.pallas import tpu_sc as plsc

BATCH = 4096
SEQ = 200
DIM = 64

_B = BATCH * SEQ
_NW = 32
_BPW = _B // _NW
_CHUNK = 800
_NCHUNK = _BPW // _CHUNK


def _embed_kernel(idx_hbm, table_hbm, out_hbm, idx_v, rows0, rows1, gsem, wsem):
    nc = 2
    wid = lax.axis_index("s") * nc + lax.axis_index("c")
    base = wid * _BPW
    pltpu.sync_copy(idx_hbm.at[pl.ds(base, _BPW)], idx_v)

    bufs = (rows0, rows1)

    def gather_start(g, buf):
        pltpu.async_copy(
            table_hbm.at[idx_v.at[pl.ds(g * _CHUNK, _CHUNK)]], buf, gsem)

    def gather_wait(buf):
        pltpu.make_async_copy(
            out_hbm.at[pl.ds(base, _CHUNK)], buf, gsem).wait()

    def write_start(g, buf):
        pltpu.async_copy(
            buf, out_hbm.at[pl.ds(base + g * _CHUNK, _CHUNK)], wsem)

    def write_wait(buf):
        pltpu.make_async_copy(
            buf, out_hbm.at[pl.ds(base, _CHUNK)], wsem).wait()

    gather_start(0, bufs[0])

    def body(g, carry):
        del carry
        for par in range(2):
            gg = g * 2 + par
            buf = bufs[par]
            other = bufs[1 - par]

            gather_wait(buf)
            write_start(gg, buf)

            @pl.when(gg >= 1)
            def _():
                write_wait(other)

            @pl.when(gg + 1 < _NCHUNK)
            def _():
                gather_start(gg + 1, other)
        return 0

    lax.fori_loop(0, _NCHUNK // 2, body, 0)
    write_wait(bufs[(_NCHUNK - 1) % 2])


@jax.jit
def kernel(input_ids, embed_tokens):
    idx = input_ids.reshape(-1).astype(jnp.int32)
    mesh = plsc.VectorSubcoreMesh(core_axis_name="c", subcore_axis_name="s")
    out = pl.kernel(
        _embed_kernel,
        mesh=mesh,
        compiler_params=pltpu.CompilerParams(use_tc_tiling_on_sc=False),
        out_type=jax.ShapeDtypeStruct((_B, DIM), jnp.float32),
        scratch_types=[
            pltpu.VMEM((_BPW,), jnp.int32),
            pltpu.VMEM((_CHUNK, DIM), jnp.float32),
            pltpu.VMEM((_CHUNK, DIM), jnp.float32),
            pltpu.SemaphoreType.DMA,
            pltpu.SemaphoreType.DMA,
        ],
    )(idx, embed_tokens)
    return out.reshape(BATCH, SEQ, DIM)
